# baseline (speedup 1.0000x reference)
import functools

import jax
import jax.numpy as jnp
from jax import lax
from jax.experimental import pallas as pl

LOOKUP_W = 101


def _hist_body(frames_ref, out_ref, *, tc):
    f = frames_ref[0]
    r = f[0, 0]
    g = f[1, 0]
    bb = f[2, 0]
    binned = ((r >> 5) << 6) + ((g >> 5) << 3) + (bb >> 5)
    hi = binned >> 7
    lo = binned & 127
    P = binned.shape[1]
    eq_lo = (lo[:, :, None] == lax.broadcasted_iota(jnp.int32, (tc, P, 128), 2))
    eq_hi = (hi[:, None, :] == lax.broadcasted_iota(jnp.int32, (tc, 4, P), 1))
    eq_lo = eq_lo.astype(jnp.bfloat16)
    eq_hi = eq_hi.astype(jnp.bfloat16)
    for i in range(tc):
        hist = lax.dot_general(
            eq_hi[i], eq_lo[i], (((1,), (0,)), ((), ())),
            preferred_element_type=jnp.float32,
        )
        n = jnp.sum(hist * hist)
        out_ref[0, i] = hist * lax.rsqrt(n)


def _tail_body(h_ref, w_ref, b_ref, out_ref, *, t):
    hb = h_ref[0]
    sims = lax.dot_general(
        hb, hb, (((1,), (1,)), ((), ())),
        preferred_element_type=jnp.float32,
        precision=lax.Precision.HIGHEST,
    )
    pad = (LOOKUP_W - 1) // 2
    width = 256
    simsp = jnp.concatenate(
        [jnp.zeros((t, pad), jnp.float32), sims,
         jnp.zeros((t, width - pad - t), jnp.float32)], axis=1)
    rows = lax.broadcasted_iota(jnp.int32, (t, width), 0)
    x = simsp
    for j in range(7):
        amt = 1 << j
        rolled = jnp.concatenate([x[:, amt:], x[:, :amt]], axis=1)
        x = jnp.where((rows & amt) != 0, rolled, x)
    gathered = x[:, :128]
    out = lax.dot_general(
        gathered, w_ref[...], (((1,), (1,)), ((), ())),
        preferred_element_type=jnp.float32,
        precision=lax.Precision.HIGHEST,
    )
    out_ref[0] = jnp.maximum(out + b_ref[...], 0.0)


def kernel(inputs, W, b):
    B, C, T, H, Wd = inputs.shape
    P = H * Wd
    tc = 10
    frames = inputs.reshape(B, C, T // tc, tc, P)
    h4 = pl.pallas_call(
        functools.partial(_hist_body, tc=tc),
        grid=(B, T // tc),
        in_specs=[pl.BlockSpec((1, 3, 1, tc, P),
                               lambda bi, ti: (bi, 0, ti, 0, 0))],
        out_specs=pl.BlockSpec((1, tc, 4, 128), lambda bi, ti: (bi, ti, 0, 0)),
        out_shape=jax.ShapeDtypeStruct((B, T, 4, 128), jnp.float32),
    )(frames)
    h = h4.reshape(B, T, 512)

    w_pad = jnp.zeros((128, 128), jnp.float32).at[:, :LOOKUP_W].set(W)
    b2 = b.reshape(1, 128)
    out = pl.pallas_call(
        functools.partial(_tail_body, t=T),
        grid=(B,),
        in_specs=[
            pl.BlockSpec((1, T, 512), lambda bi: (bi, 0, 0)),
            pl.BlockSpec((128, 128), lambda bi: (0, 0)),
            pl.BlockSpec((1, 128), lambda bi: (0, 0)),
        ],
        out_specs=pl.BlockSpec((1, T, 128), lambda bi: (bi, 0, 0)),
        out_shape=jax.ShapeDtypeStruct((B, T, 128), jnp.float32),
    )(h, w_pad, b2)
    return out


# speedup vs baseline: 2.3853x; 2.3853x over previous
import functools

import jax
import jax.numpy as jnp
from jax import lax
from jax.experimental import pallas as pl
from jax.experimental.pallas import tpu as pltpu
from jax.experimental.pallas import tpu_sc as plsc

LOOKUP_W = 101
_GF = 4
_NW = 32


def _tail_body(h_ref, w_ref, b_ref, out_ref, *, t):
    x = h_ref[0]
    h = (x[:, 0:512] + x[:, 512:1024]
         + x[:, 1024:1536] + x[:, 1536:2048])
    n = jnp.sum(h * h, axis=1, keepdims=True)
    hb = h * lax.rsqrt(n)
    sims = lax.dot_general(
        hb, hb, (((1,), (1,)), ((), ())),
        preferred_element_type=jnp.float32,
        precision=lax.Precision.HIGHEST,
    )
    pad = (LOOKUP_W - 1) // 2
    width = 256
    simsp = jnp.concatenate(
        [jnp.zeros((t, pad), jnp.float32), sims,
         jnp.zeros((t, width - pad - t), jnp.float32)], axis=1)
    rows = lax.broadcasted_iota(jnp.int32, (t, width), 0)
    x = simsp
    for j in range(7):
        amt = 1 << j
        rolled = jnp.concatenate([x[:, amt:], x[:, :amt]], axis=1)
        x = jnp.where((rows & amt) != 0, rolled, x)
    gathered = x[:, :128]
    out = lax.dot_general(
        gathered, w_ref[...], (((1,), (1,)), ((), ())),
        preferred_element_type=jnp.float32,
        precision=lax.Precision.HIGHEST,
    )
    out_ref[0] = jnp.maximum(out + b_ref[...], 0.0)


def kernel(inputs, W, b):
    B, C, T, H, Wd = inputs.shape
    P = H * Wd
    frames = inputs.reshape(B, C, T * P)
    G = (B * T) // _GF
    ng = -(-G // _NW)
    tpb = T // _GF
    gsz = _GF * P
    bsz = C * gsz
    hsz = 16 * 512

    mesh = plsc.VectorSubcoreMesh(core_axis_name="c", subcore_axis_name="s")

    @functools.partial(
        pl.kernel,
        mesh=mesh,
        compiler_params=pltpu.CompilerParams(
            use_tc_tiling_on_sc=False, needs_layout_passes=False),
        out_type=jax.ShapeDtypeStruct((B, T, _GF * 512), jnp.float32),
        scratch_types=[
            pltpu.VMEM((2 * bsz,), jnp.int32),
            pltpu.VMEM((2 * hsz,), jnp.float32),
            pltpu.SemaphoreType.DMA,
            pltpu.SemaphoreType.DMA,
        ],
    )
    def sc_hist(frames_hbm, out_hbm, buf, hist, sem_in, sem_out):
        w = lax.axis_index("s") * 2 + lax.axis_index("c")
        lane = lax.iota(jnp.int32, 16)
        base_v = (lane >> 2) * P + (lane & 3)
        lane512 = lane * 512
        ones = jnp.full((16,), 1.0, jnp.float32)
        zeros = jnp.full((16,), 0.0, jnp.float32)

        def group_coords(gi):
            g = jnp.minimum(w + _NW * gi, G - 1)
            bi = g // tpb
            t0 = (g % tpb) * _GF
            return bi, t0

        def fire_in(gi, half):
            bi, t0 = group_coords(gi)
            return [
                pltpu.async_copy(
                    frames_hbm.at[bi, ch, pl.ds(t0 * P, gsz)],
                    buf.at[pl.ds(half * bsz + ch * gsz, gsz)], sem_in)
                for ch in range(C)
            ]

        pending = fire_in(0, 0)
        for gi in range(ng):
            half = gi % 2
            bi, t0 = group_coords(gi)
            def zbody(j, _):
                for r in range(16):
                    hist[pl.ds(half * hsz + j * 256 + r * 16, 16)] = zeros
                return 0
            lax.fori_loop(0, hsz // 256, zbody, 0)
            for hdl in pending:
                hdl.wait()
            if gi + 1 < ng:
                pending = fire_in(gi + 1, (gi + 1) % 2)
            hoff = half * hsz

            def pbody(j, _):
                for k in range(4):
                    s = half * bsz + j * 16 + k * 4
                    r = plsc.load_gather(buf, [base_v + s])
                    g_ = plsc.load_gather(buf, [base_v + (s + gsz)])
                    bb = plsc.load_gather(buf, [base_v + (s + 2 * gsz)])
                    binv = ((r >> 5) << 6) + ((g_ >> 5) << 3) + (bb >> 5)
                    plsc.addupdate_scatter(hist, [binv + lane512 + hoff], ones)
                return 0
            lax.fori_loop(0, P // 16, pbody, 0)

            outs = [
                pltpu.async_copy(
                    hist.at[pl.ds(half * hsz + f * _GF * 512, _GF * 512)],
                    out_hbm.at[bi, t0 + f], sem_out)
                for f in range(_GF)
            ]
            for hdl in outs:
                hdl.wait()

    h4 = sc_hist(frames)

    w_pad = jnp.zeros((128, 128), jnp.float32).at[:, :LOOKUP_W].set(W)
    b2 = b.reshape(1, 128)
    out = pl.pallas_call(
        functools.partial(_tail_body, t=T),
        grid=(B,),
        in_specs=[
            pl.BlockSpec((1, T, _GF * 512), lambda bi: (bi, 0, 0)),
            pl.BlockSpec((128, 128), lambda bi: (0, 0)),
            pl.BlockSpec((1, 128), lambda bi: (0, 0)),
        ],
        out_specs=pl.BlockSpec((1, T, 128), lambda bi: (bi, 0, 0)),
        out_shape=jax.ShapeDtypeStruct((B, T, 128), jnp.float32),
    )(h4, w_pad, b2)
    return out


# speedup vs baseline: 2.4957x; 1.0463x over previous
import functools

import jax
import jax.numpy as jnp
from jax import lax
from jax.experimental import pallas as pl
from jax.experimental.pallas import tpu as pltpu
from jax.experimental.pallas import tpu_sc as plsc

LOOKUP_W = 101
_GF = 4
_NW = 32


def _tail_body(h_ref, w_ref, b_ref, out_ref, *, t):
    x = h_ref[0]
    h = (x[:, 0:512] + x[:, 512:1024]
         + x[:, 1024:1536] + x[:, 1536:2048])
    n = jnp.sum(h * h, axis=1, keepdims=True)
    hb = h * lax.rsqrt(n)
    sims = lax.dot_general(
        hb, hb, (((1,), (1,)), ((), ())),
        preferred_element_type=jnp.float32,
    )
    pad = (LOOKUP_W - 1) // 2
    width = 256
    simsp = jnp.concatenate(
        [jnp.zeros((t, pad), jnp.float32), sims,
         jnp.zeros((t, width - pad - t), jnp.float32)], axis=1)
    rows = lax.broadcasted_iota(jnp.int32, (t, width), 0)
    x = simsp
    for j in range(7):
        amt = 1 << j
        rolled = jnp.concatenate([x[:, amt:], x[:, :amt]], axis=1)
        x = jnp.where((rows & amt) != 0, rolled, x)
    gathered = x[:, :128]
    out = lax.dot_general(
        gathered, w_ref[...], (((1,), (1,)), ((), ())),
        preferred_element_type=jnp.float32,
    )
    out_ref[0] = jnp.maximum(out + b_ref[...], 0.0)


def kernel(inputs, W, b):
    B, C, T, H, Wd = inputs.shape
    P = H * Wd
    frames = inputs.reshape(B, C, T * P)
    G = (B * T) // _GF
    ng = -(-G // _NW)
    tpb = T // _GF
    gsz = _GF * P
    bsz = C * gsz
    hsz = 16 * 512

    mesh = plsc.VectorSubcoreMesh(core_axis_name="c", subcore_axis_name="s")

    @functools.partial(
        pl.kernel,
        mesh=mesh,
        compiler_params=pltpu.CompilerParams(
            use_tc_tiling_on_sc=False, needs_layout_passes=False),
        out_type=jax.ShapeDtypeStruct((B, T, _GF * 512), jnp.float32),
        scratch_types=[
            pltpu.VMEM((2 * bsz,), jnp.int32),
            pltpu.VMEM((2 * hsz,), jnp.float32),
            pltpu.SemaphoreType.DMA,
            pltpu.SemaphoreType.DMA,
        ],
    )
    def sc_hist(frames_hbm, out_hbm, buf, hist, sem_in, sem_out):
        w = lax.axis_index("s") * 2 + lax.axis_index("c")
        lane = lax.iota(jnp.int32, 16)
        base_v = (lane >> 2) * P + (lane & 3)
        lane512 = lane * 512
        ones = jnp.full((16,), 1.0, jnp.float32)
        zeros = jnp.full((16,), 0.0, jnp.float32)

        def group_coords(gi):
            g = jnp.minimum(w + _NW * gi, G - 1)
            bi = g // tpb
            t0 = (g % tpb) * _GF
            return bi, t0

        def fire_in(gi, half):
            bi, t0 = group_coords(gi)
            return [
                pltpu.async_copy(
                    frames_hbm.at[bi, ch, pl.ds(t0 * P, gsz)],
                    buf.at[pl.ds(half * bsz + ch * gsz, gsz)], sem_in)
                for ch in range(C)
            ]

        pending = fire_in(0, 0)
        outs_pending = {0: [], 1: []}
        for gi in range(ng):
            half = gi % 2
            bi, t0 = group_coords(gi)
            for hdl in outs_pending[half]:
                hdl.wait()
            def zbody(j, _):
                for r in range(16):
                    hist[pl.ds(half * hsz + j * 256 + r * 16, 16)] = zeros
                return 0
            lax.fori_loop(0, hsz // 256, zbody, 0)
            for hdl in pending:
                hdl.wait()
            if gi + 1 < ng:
                pending = fire_in(gi + 1, (gi + 1) % 2)
            hoff = half * hsz

            def pbody(j, _):
                for k in range(12):
                    s = half * bsz + j * 48 + k * 4
                    r = plsc.load_gather(buf, [base_v + s])
                    g_ = plsc.load_gather(buf, [base_v + (s + gsz)])
                    bb = plsc.load_gather(buf, [base_v + (s + 2 * gsz)])
                    binv = ((r >> 5) << 6) + ((g_ >> 5) << 3) + (bb >> 5)
                    plsc.addupdate_scatter(hist, [binv + lane512 + hoff], ones)
                return 0
            lax.fori_loop(0, P // 48, pbody, 0)

            outs_pending[half] = [
                pltpu.async_copy(
                    hist.at[pl.ds(half * hsz + f * _GF * 512, _GF * 512)],
                    out_hbm.at[bi, t0 + f], sem_out)
                for f in range(_GF)
            ]
        for hdl in outs_pending[0] + outs_pending[1]:
            hdl.wait()

    h4 = sc_hist(frames)

    w_pad = jnp.zeros((128, 128), jnp.float32).at[:, :LOOKUP_W].set(W)
    b2 = b.reshape(1, 128)
    out = pl.pallas_call(
        functools.partial(_tail_body, t=T),
        grid=(B,),
        in_specs=[
            pl.BlockSpec((1, T, _GF * 512), lambda bi: (bi, 0, 0)),
            pl.BlockSpec((128, 128), lambda bi: (0, 0)),
            pl.BlockSpec((1, 128), lambda bi: (0, 0)),
        ],
        out_specs=pl.BlockSpec((1, T, 128), lambda bi: (bi, 0, 0)),
        out_shape=jax.ShapeDtypeStruct((B, T, 128), jnp.float32),
    )(h4, w_pad, b2)
    return out
